# baseline (speedup 1.0000x reference)
import functools

import jax
import jax.numpy as jnp
from jax import lax
from jax.experimental import pallas as pl
from jax.experimental.pallas import tpu as pltpu
from jax.experimental.pallas import tpu_sc as plsc

B = 16384
N_NUM = 13
N_CAT = 26
VOCAB = 100001
EMB = 50
PITCH = 56
OUT_W = N_NUM + N_CAT * EMB
OUT_PAD = 1328

_INFO = plsc.get_sparse_core_info()
NC = _INFO.num_cores
NS = _INFO.num_subcores
NW = NC * NS

CB = 16
N_CHUNK_TOT = B // CB
N_CHUNKS = N_CHUNK_TOT // NW
DESC_PER_CHUNK = CB * N_CAT * 2
G = 13
GW = DESC_PER_CHUNK // G


def _project(idx3, q3, x3, flat_tables):
    mesh = plsc.VectorSubcoreMesh(core_axis_name="c", subcore_axis_name="s")

    @functools.partial(
        pl.kernel,
        mesh=mesh,
        out_type=jax.ShapeDtypeStruct((B, OUT_PAD), jnp.float32),
        scratch_types=[
            pltpu.VMEM((G, GW), jnp.int32),
            pltpu.VMEM((CB * N_CAT, 16), jnp.int32),
            pltpu.VMEM((CB, 16), jnp.float32),
            pltpu.VMEM((DESC_PER_CHUNK + 2, EMB), jnp.float32),
            pltpu.VMEM((CB, OUT_PAD), jnp.float32),
            pltpu.SemaphoreType.DMA,
        ],
        compiler_params=pltpu.CompilerParams(
            use_tc_tiling_on_sc=False, needs_layout_passes=False
        ),
    )
    def k(idx_hbm, q_hbm, x_hbm, table_hbm, out_hbm,
          idx_v, q_v, x_v, rows_v, outbuf, sem):
        wid = lax.axis_index("s") * NC + lax.axis_index("c")
        lanes = lax.iota(jnp.int32, 16)
        zeros = lanes - lanes

        def body(ci, _):
            c = wid * N_CHUNKS + ci
            pltpu.sync_copy(idx_hbm.at[c], idx_v)
            pltpu.sync_copy(q_hbm.at[c], q_v)
            pltpu.sync_copy(x_hbm.at[c], x_v)
            copies = [
                pltpu.async_copy(
                    table_hbm.at[idx_v.at[j]],
                    rows_v.at[pl.ds(j * GW, GW)],
                    sem,
                )
                for j in range(G)
            ]
            for cp in copies:
                cp.wait()

            def row_body(m, _):
                outbuf[m, pl.ds(0, 16)] = x_v[m]
                pbase = m * N_CAT
                for f in range(N_CAT):
                    q0 = q_v[pbase + f] + lanes
                    d0 = N_NUM + EMB * f
                    for kk in range(4):
                        v = plsc.load_gather(rows_v, [zeros, q0 + 16 * kk])
                        outbuf[m, pl.ds(d0 + 16 * kk, 16)] = v
                return 0

            lax.fori_loop(0, CB, row_body, 0)
            pltpu.sync_copy(outbuf, out_hbm.at[pl.ds(c * CB, CB)])
            return 0

        lax.fori_loop(0, N_CHUNKS, body, 0)

    return k(idx3, q3, x3, flat_tables)


def kernel(x_num, x_cat, tables):
    flat_tables = tables.reshape(N_CAT * VOCAB, EMB)
    r = x_cat + jnp.arange(N_CAT, dtype=jnp.int32) * VOCAB
    i0 = (PITCH * r) // EMB
    ph = PITCH * r - EMB * i0
    idx2 = jnp.stack([i0, i0 + 1], axis=-1)
    idx3 = idx2.reshape(N_CHUNK_TOT, G, GW)
    m_in_chunk = (jnp.arange(B, dtype=jnp.int32) % CB)[:, None]
    fcol = jnp.arange(N_CAT, dtype=jnp.int32)[None, :]
    qbase = EMB * (2 * N_CAT * m_in_chunk + 2 * fcol) + ph
    q3 = jnp.broadcast_to(qbase[:, :, None], (B, N_CAT, 16))
    q3 = q3.reshape(N_CHUNK_TOT, CB * N_CAT, 16)
    x3 = jnp.pad(x_num, ((0, 0), (0, 3))).reshape(N_CHUNK_TOT, CB, 16)
    out = _project(idx3, q3, x3, flat_tables)
    return out[:, :OUT_W]
